# Initial kernel scaffold; baseline (speedup 1.0000x reference)
import jax
import jax.numpy as jnp
from jax.experimental import pallas as pl


def kernel(x, embed_table, router_w):
    raise NotImplementedError("write your pallas kernel here")


# speedup vs baseline: 1.0438x; 1.0438x over previous
import jax
import jax.numpy as jnp
from jax import lax
from jax.experimental import pallas as pl
from jax.experimental.pallas import tpu as pltpu
from jax.experimental.pallas import tpu_sc as plsc

EMB = 64
NC, NS = 2, 16
NW = NC * NS
CHUNK = 1024
MM_BLK = 4096


def _gather_body(table_hbm, idx_hbm, out_hbm, idx_v, rows_v, sem):
    wid = lax.axis_index("s") * NC + lax.axis_index("c")
    n_per_w = idx_v.shape[0]
    base = wid * n_per_w
    pltpu.sync_copy(idx_hbm.at[pl.ds(base, n_per_w)], idx_v)
    n_chunks = n_per_w // CHUNK

    def body(c, carry):
        off = c * CHUNK
        pltpu.async_copy(
            table_hbm.at[idx_v.at[pl.ds(off, CHUNK)]], rows_v, sem
        ).wait()
        pltpu.sync_copy(rows_v, out_hbm.at[pl.ds(base + off, CHUNK)])
        return carry

    lax.fori_loop(0, n_chunks, body, 0)


def _sc_gather(table, idx_flat):
    n = idx_flat.shape[0]
    n_per_w = n // NW
    mesh = plsc.VectorSubcoreMesh(core_axis_name="c", subcore_axis_name="s")
    k = pl.kernel(
        _gather_body,
        out_type=jax.ShapeDtypeStruct((n, EMB), jnp.float32),
        mesh=mesh,
        scratch_types=[
            pltpu.VMEM((n_per_w,), jnp.int32),
            pltpu.VMEM((CHUNK, EMB), jnp.float32),
            pltpu.SemaphoreType.DMA,
        ],
        compiler_params=pltpu.CompilerParams(use_tc_tiling_on_sc=False),
    )
    return k(table, idx_flat)


def _mm_body(h_ref, w_ref, out_ref):
    out_ref[...] = lax.dot_general(
        h_ref[...], w_ref[...], (((1,), (1,)), ((), ())),
        preferred_element_type=jnp.float32,
    )


def _tc_logits(h_flat, w):
    n = h_flat.shape[0]
    n_exp = w.shape[0]
    return pl.pallas_call(
        _mm_body,
        grid=(n // MM_BLK,),
        in_specs=[
            pl.BlockSpec((MM_BLK, EMB), lambda i: (i, 0)),
            pl.BlockSpec(w.shape, lambda i: (0, 0)),
        ],
        out_specs=pl.BlockSpec((MM_BLK, n_exp), lambda i: (i, 0)),
        out_shape=jax.ShapeDtypeStruct((n, n_exp), jnp.float32),
    )(h_flat, w)


def kernel(x, embed_table, router_w):
    b, l = x.shape
    idx_flat = x.reshape(-1).astype(jnp.int32)
    h_flat = _sc_gather(embed_table, idx_flat)
    logits_flat = _tc_logits(h_flat, router_w)
    return (
        h_flat.reshape(b, l, EMB),
        logits_flat.reshape(b, l, router_w.shape[0]),
    )


# speedup vs baseline: 1.0650x; 1.0204x over previous
import jax
import jax.numpy as jnp
from jax import lax
from jax.experimental import pallas as pl
from jax.experimental.pallas import tpu as pltpu
from jax.experimental.pallas import tpu_sc as plsc

EMB = 64
NC, NS = 2, 16
NW = NC * NS
CHUNK = 1024
NB = 32


def _gather_body(table_hbm, idx_hbm, out_hbm, idx_v, rows_v, sem):
    wid = lax.axis_index("s") * NC + lax.axis_index("c")
    n_per_w = idx_v.shape[0]
    base = wid * n_per_w
    pltpu.sync_copy(idx_hbm.at[pl.ds(base, n_per_w)], idx_v)
    n_chunks = n_per_w // CHUNK

    def body(c, carry):
        off = c * CHUNK
        pltpu.async_copy(
            table_hbm.at[idx_v.at[pl.ds(off, CHUNK)]], rows_v, sem
        ).wait()
        pltpu.sync_copy(rows_v, out_hbm.at[pl.ds(base + off, CHUNK)])
        return carry

    lax.fori_loop(0, n_chunks, body, 0)


def _sc_gather(table, idx_flat):
    n = idx_flat.shape[0]
    n_per_w = n // NW
    mesh = plsc.VectorSubcoreMesh(core_axis_name="c", subcore_axis_name="s")
    k = pl.kernel(
        _gather_body,
        out_type=jax.ShapeDtypeStruct((n, EMB), jnp.float32),
        mesh=mesh,
        scratch_types=[
            pltpu.VMEM((n_per_w,), jnp.int32),
            pltpu.VMEM((CHUNK, EMB), jnp.float32),
            pltpu.SemaphoreType.DMA,
        ],
        compiler_params=pltpu.CompilerParams(use_tc_tiling_on_sc=False),
    )
    return k(table, idx_flat)


def _unpair(p, out_shape):
    stacked = jnp.stack([p[:, :EMB], p[:, EMB:]], axis=1)
    return stacked.reshape(out_shape)


def _tc_body(h_ref, w2_ref, h_out_ref, lg_out_ref):
    h_blk = h_ref[...]
    p = jnp.dot(h_blk, w2_ref[...], preferred_element_type=jnp.float32)
    lg_out_ref[...] = _unpair(p, lg_out_ref.shape)
    h_out_ref[...] = _unpair(h_blk, h_out_ref.shape)


def _tc_logits_and_h(h128, w2, b, l):
    rows_per_blk = NB * l // 2
    return pl.pallas_call(
        _tc_body,
        grid=(b // NB,),
        in_specs=[
            pl.BlockSpec((rows_per_blk, 2 * EMB), lambda i: (i, 0)),
            pl.BlockSpec((2 * EMB, 2 * EMB), lambda i: (0, 0)),
        ],
        out_specs=[
            pl.BlockSpec((NB, l, EMB), lambda i: (i, 0, 0)),
            pl.BlockSpec((NB, l, EMB), lambda i: (i, 0, 0)),
        ],
        out_shape=[
            jax.ShapeDtypeStruct((b, l, EMB), jnp.float32),
            jax.ShapeDtypeStruct((b, l, EMB), jnp.float32),
        ],
    )(h128, w2)


def kernel(x, embed_table, router_w):
    b, l = x.shape
    n_exp = router_w.shape[0]
    idx_flat = x.reshape(-1).astype(jnp.int32)
    h_flat = _sc_gather(embed_table, idx_flat)
    h128 = h_flat.reshape(b * l // 2, 2 * EMB)
    wt = router_w.T
    zero = jnp.zeros((EMB, n_exp), jnp.float32)
    w2 = jnp.concatenate(
        [
            jnp.concatenate([wt, zero], axis=1),
            jnp.concatenate([zero, wt], axis=1),
        ],
        axis=0,
    )
    h_out, lg_out = _tc_logits_and_h(h128, w2, b, l)
    return (h_out, lg_out)


# speedup vs baseline: 1.3538x; 1.2711x over previous
import jax
import jax.numpy as jnp
from jax import lax
from jax.experimental import pallas as pl
from jax.experimental.pallas import tpu as pltpu
from jax.experimental.pallas import tpu_sc as plsc

EMB = 64
NC, NS = 2, 16
NW = NC * NS
CHUNK = 1024
NB = 32


def _gather_body(table_hbm, idx_hbm, out_hbm, idx_v, rows_v, sem):
    wid = lax.axis_index("s") * NC + lax.axis_index("c")
    n_per_w = idx_v.shape[0]
    base = wid * n_per_w
    pltpu.sync_copy(idx_hbm.at[pl.ds(base, n_per_w)], idx_v)
    n_chunks = n_per_w // CHUNK

    def body(c, carry):
        off = c * CHUNK
        pltpu.async_copy(
            table_hbm.at[idx_v.at[pl.ds(off, CHUNK)]], rows_v, sem
        ).wait()
        pltpu.sync_copy(rows_v, out_hbm.at[pl.ds(base + off, CHUNK)])
        return carry

    lax.fori_loop(0, n_chunks, body, 0)


def _sc_gather(table, idx_flat):
    n = idx_flat.shape[0]
    n_per_w = n // NW
    mesh = plsc.VectorSubcoreMesh(core_axis_name="c", subcore_axis_name="s")
    k = pl.kernel(
        _gather_body,
        out_type=jax.ShapeDtypeStruct((n, EMB), jnp.float32),
        mesh=mesh,
        scratch_types=[
            pltpu.VMEM((n_per_w,), jnp.int32),
            pltpu.VMEM((CHUNK, EMB), jnp.float32),
            pltpu.SemaphoreType.DMA,
        ],
        compiler_params=pltpu.CompilerParams(use_tc_tiling_on_sc=False),
    )
    return k(table, idx_flat)


MM_BLK = 4096


def _tc_body(h_ref, w2_ref, lg_ref):
    lg_ref[...] = jnp.dot(
        h_ref[...], w2_ref[...], preferred_element_type=jnp.float32
    )


def _tc_logits(h128, w2):
    n = h128.shape[0]
    return pl.pallas_call(
        _tc_body,
        grid=(n // MM_BLK,),
        in_specs=[
            pl.BlockSpec((MM_BLK, 2 * EMB), lambda i: (i, 0)),
            pl.BlockSpec((2 * EMB, 2 * EMB), lambda i: (0, 0)),
        ],
        out_specs=pl.BlockSpec((MM_BLK, 2 * EMB), lambda i: (i, 0)),
        out_shape=jax.ShapeDtypeStruct((n, 2 * EMB), jnp.float32),
    )(h128, w2)


def kernel(x, embed_table, router_w):
    b, l = x.shape
    n_exp = router_w.shape[0]
    idx_flat = x.reshape(-1).astype(jnp.int32)
    h_flat = _sc_gather(embed_table, idx_flat)
    h128 = h_flat.reshape(b * l // 2, 2 * EMB)
    wt = router_w.T
    zero = jnp.zeros((EMB, n_exp), jnp.float32)
    w2 = jnp.concatenate(
        [
            jnp.concatenate([wt, zero], axis=1),
            jnp.concatenate([zero, wt], axis=1),
        ],
        axis=0,
    )
    lg128 = _tc_logits(h128, w2)
    return (
        h_flat.reshape(b, l, EMB),
        lg128.reshape(b, l, n_exp),
    )
